# baseline (speedup 1.0000x reference)
import functools

import jax
import jax.numpy as jnp
from jax import lax
from jax.experimental import pallas as pl
from jax.experimental.pallas import tpu as pltpu
from jax.experimental.pallas import tpu_sc as plsc

N_NODES = 10000
IN_FEAT = 128
OUT_FEAT = 128
NUM_RELS = 16
NUM_BASES = 64
SUBMAT = 2

CHUNK = 128
N_PIECES = 4
H_ROWS = 10240
MM_ROWS = 400
CB_ROWS = 1000


def _transform_body(x_ref, w_ref, y_ref):
    y_ref[...] = jnp.dot(x_ref[...], w_ref[...], preferred_element_type=jnp.float32)


def _combine_body(a_ref, b_ref, bias_ref, o_ref):
    o_ref[...] = a_ref[0] + b_ref[0] + bias_ref[...]


def kernel(x, edge_index, etypes, weight, h_bias):
    n_nodes, in_feat = x.shape
    src = edge_index[0].astype(jnp.int32)
    dst = edge_index[1].astype(jnp.int32)
    et = etypes.astype(jnp.int32)
    n_edges = src.shape[0]

    blocks = weight.reshape(NUM_RELS, NUM_BASES, SUBMAT, SUBMAT)
    eye = jnp.eye(NUM_BASES, dtype=weight.dtype)
    wd = jnp.einsum("rbuv,bc->rbucv", blocks, eye)
    w2 = wd.reshape(NUM_RELS, IN_FEAT, OUT_FEAT).transpose(1, 0, 2).reshape(
        IN_FEAT, NUM_RELS * OUT_FEAT)
    y2 = pl.pallas_call(
        _transform_body,
        grid=(n_nodes // MM_ROWS,),
        in_specs=[
            pl.BlockSpec((MM_ROWS, IN_FEAT), lambda i: (i, 0)),
            pl.BlockSpec((IN_FEAT, NUM_RELS * OUT_FEAT), lambda i: (0, 0)),
        ],
        out_specs=pl.BlockSpec((MM_ROWS, NUM_RELS * OUT_FEAT), lambda i: (i, 0)),
        out_shape=jax.ShapeDtypeStruct((n_nodes, NUM_RELS * OUT_FEAT), jnp.float32),
    )(x, w2)
    y_rows = y2.reshape(n_nodes * NUM_RELS, OUT_FEAT)

    e_pad = ((n_edges + 32 * CHUNK - 1) // (32 * CHUNK)) * (32 * CHUNK)
    pad = e_pad - n_edges
    src_p = jnp.concatenate([src, jnp.zeros((pad,), jnp.int32)])
    et_p = jnp.concatenate([et, jnp.zeros((pad,), jnp.int32)])
    pad_rows = n_nodes + jnp.arange(pad, dtype=jnp.int32) % (H_ROWS - n_nodes)
    dst_p = jnp.concatenate([dst, pad_rows])

    mesh = plsc.VectorSubcoreMesh(core_axis_name="c", subcore_axis_name="s")
    n_chunks = e_pad // (32 * CHUNK)
    per_w = n_chunks * CHUNK
    rows_per_tile = H_ROWS // 16

    src3 = src_p.reshape(32, per_w)
    et3 = et_p.reshape(32, per_w)
    dst3 = dst_p.reshape(32, n_chunks, CHUNK)

    def sc_body(y_hbm, src_hbm, et_hbm, dst_hbm, out_hbm,
                src_v, et_v, key_v, dst_v, rows0_v, rows1_v,
                zrow_v, h_sh, sem0, sem1, sem2, sem3):
        cid = lax.axis_index("c")
        sid = lax.axis_index("s")
        wid = sid * 2 + cid

        for i in range(16):
            for j in range(OUT_FEAT // 16):
                zrow_v[i, pl.ds(j * 16, 16)] = jnp.zeros((16,), jnp.float32)

        def zero_body(k, _):
            pltpu.sync_copy(zrow_v, h_sh.at[pl.ds(sid * rows_per_tile + k * 16, 16)])
            return _
        lax.fori_loop(0, rows_per_tile // 16, zero_body, None)

        pltpu.sync_copy(dst_hbm.at[wid], dst_v)
        piece = per_w // N_PIECES

        def piece_body(p, _):
            pltpu.sync_copy(src_hbm.at[wid, pl.ds(p * piece, piece)], src_v)
            pltpu.sync_copy(et_hbm.at[wid, pl.ds(p * piece, piece)], et_v)

            def key_body(i, _):
                sl = pl.ds(i * 16, 16)
                key_v[pl.ds(p * piece + i * 16, 16)] = (
                    src_v[sl] * NUM_RELS + et_v[sl])
                return _
            lax.fori_loop(0, piece // 16, key_body, None)
            return _
        lax.fori_loop(0, N_PIECES, piece_body, None)
        plsc.subcore_barrier()

        bufs = (rows0_v, rows1_v)
        gsems = (sem0, sem1)
        ssems = (sem2, sem3)

        def start_gather(t, b):
            pltpu.async_copy(y_hbm.at[key_v.at[pl.ds(t * CHUNK, CHUNK)]],
                             bufs[b], gsems[b])

        def wait_gather(b):
            pltpu.make_async_copy(y_hbm.at[key_v.at[pl.ds(0, CHUNK)]],
                                  bufs[b], gsems[b]).wait()

        def convert(b):
            bf = bfbufs[b]
            fb = bufs[b]

            def conv_row(e, _):
                for cc in range(OUT_FEAT // 32):
                    w = bf[e, pl.ds(cc * 16, 16)]
                    f_even = plsc.bitcast(w << 16, jnp.float32)
                    f_odd = plsc.bitcast(w & jnp.int32(-65536), jnp.float32)
                    fb[e, pl.ds(cc * 32, 16)] = f_even
                    fb[e, pl.ds(cc * 32 + 16, 16)] = f_odd
                return _
            lax.fori_loop(0, CHUNK, conv_row, None)

        def start_scatter(t, b):
            pltpu.async_copy(bufs[b], h_sh.at[dst_v.at[t]], ssems[b],
                             add=True)

        def wait_scatter(b):
            pltpu.make_async_copy(bufs[b], h_sh.at[dst_v.at[0]],
                                  ssems[b]).wait()

        start_gather(0, 0)

        @pl.when(1 < n_chunks)
        def _():
            start_gather(1, 1)

        def pair_body(i, _):
            t0 = i * 2
            t1 = t0 + 1
            wait_gather(0)
            start_scatter(t0, 0)

            @pl.when(t1 < n_chunks)
            def _():
                wait_gather(1)
                start_scatter(t1, 1)
            wait_scatter(0)

            @pl.when(t0 + 2 < n_chunks)
            def _():
                start_gather(t0 + 2, 0)

            @pl.when(t1 < n_chunks)
            def _():
                wait_scatter(1)

                @pl.when(t1 + 2 < n_chunks)
                def _():
                    start_gather(t1 + 2, 1)
            return _
        lax.fori_loop(0, (n_chunks + 1) // 2, pair_body, None)
        plsc.subcore_barrier()

        pltpu.sync_copy(h_sh.at[pl.ds(sid * rows_per_tile, rows_per_tile)],
                        out_hbm.at[cid, pl.ds(sid * rows_per_tile, rows_per_tile)])

    partials = pl.kernel(
        sc_body,
        out_type=jax.ShapeDtypeStruct((2, H_ROWS, OUT_FEAT), jnp.float32),
        mesh=mesh,
        scratch_types=[
            pltpu.VMEM((per_w // N_PIECES,), jnp.int32),
            pltpu.VMEM((per_w // N_PIECES,), jnp.int32),
            pltpu.VMEM((per_w,), jnp.int32),
            pltpu.VMEM((n_chunks, CHUNK), jnp.int32),
            pltpu.VMEM((CHUNK, OUT_FEAT), jnp.float32),
            pltpu.VMEM((CHUNK, OUT_FEAT), jnp.float32),
            pltpu.VMEM((16, OUT_FEAT), jnp.float32),
            pltpu.VMEM_SHARED((H_ROWS, OUT_FEAT), jnp.float32),
            pltpu.SemaphoreType.DMA,
            pltpu.SemaphoreType.DMA,
            pltpu.SemaphoreType.DMA,
            pltpu.SemaphoreType.DMA,
        ],
    )(y_rows, src3, et3, dst3)

    bias2 = h_bias.reshape(1, OUT_FEAT)
    out = pl.pallas_call(
        _combine_body,
        grid=(n_nodes // CB_ROWS,),
        in_specs=[
            pl.BlockSpec((1, CB_ROWS, OUT_FEAT), lambda i: (0, i, 0)),
            pl.BlockSpec((1, CB_ROWS, OUT_FEAT), lambda i: (1, i, 0)),
            pl.BlockSpec((1, OUT_FEAT), lambda i: (0, 0)),
        ],
        out_specs=pl.BlockSpec((CB_ROWS, OUT_FEAT), lambda i: (i, 0)),
        out_shape=jax.ShapeDtypeStruct((n_nodes, OUT_FEAT), jnp.float32),
    )(partials, partials, bias2)
    return out


# speedup vs baseline: 1.0167x; 1.0167x over previous
import functools

import jax
import jax.numpy as jnp
from jax import lax
from jax.experimental import pallas as pl
from jax.experimental.pallas import tpu as pltpu
from jax.experimental.pallas import tpu_sc as plsc

N_NODES = 10000
IN_FEAT = 128
OUT_FEAT = 128
NUM_RELS = 16
NUM_BASES = 64
SUBMAT = 2

CHUNK = 128
N_PIECES = 4
H_ROWS = 10240
MM_ROWS = 400
CB_ROWS = 1000


def _transform_body(x_ref, w_ref, y_ref):
    y_ref[...] = jnp.dot(x_ref[...], w_ref[...], preferred_element_type=jnp.float32)


def _combine_body(a_ref, b_ref, bias_ref, o_ref):
    o_ref[...] = a_ref[0] + b_ref[0] + bias_ref[...]


def kernel(x, edge_index, etypes, weight, h_bias):
    n_nodes, in_feat = x.shape
    src = edge_index[0].astype(jnp.int32)
    dst = edge_index[1].astype(jnp.int32)
    et = etypes.astype(jnp.int32)
    n_edges = src.shape[0]

    blocks = weight.reshape(NUM_RELS, NUM_BASES, SUBMAT, SUBMAT)
    eye = jnp.eye(NUM_BASES, dtype=weight.dtype)
    wd = jnp.einsum("rbuv,bc->rbucv", blocks, eye)
    w2 = wd.reshape(NUM_RELS, IN_FEAT, OUT_FEAT).transpose(1, 0, 2).reshape(
        IN_FEAT, NUM_RELS * OUT_FEAT)

    y2 = pl.pallas_call(
        _transform_body,
        grid=(n_nodes // MM_ROWS,),
        in_specs=[
            pl.BlockSpec((MM_ROWS, IN_FEAT), lambda i: (i, 0)),
            pl.BlockSpec((IN_FEAT, NUM_RELS * OUT_FEAT), lambda i: (0, 0)),
        ],
        out_specs=pl.BlockSpec((MM_ROWS, NUM_RELS * OUT_FEAT), lambda i: (i, 0)),
        out_shape=jax.ShapeDtypeStruct((n_nodes, NUM_RELS * OUT_FEAT), jnp.float32),
    )(x, w2)
    y_rows = y2.reshape(n_nodes * NUM_RELS, OUT_FEAT)

    e_pad = ((n_edges + 32 * CHUNK - 1) // (32 * CHUNK)) * (32 * CHUNK)
    pad = e_pad - n_edges
    src_p = jnp.concatenate([src, jnp.zeros((pad,), jnp.int32)])
    et_p = jnp.concatenate([et, jnp.zeros((pad,), jnp.int32)])
    dst_p = jnp.concatenate([dst, jnp.full((pad,), n_nodes, jnp.int32)])

    mesh = plsc.VectorSubcoreMesh(core_axis_name="c", subcore_axis_name="s")
    n_chunks = e_pad // (32 * CHUNK)
    per_w = n_chunks * CHUNK
    rows_per_tile = H_ROWS // 16

    src3 = src_p.reshape(32, per_w)
    et3 = et_p.reshape(32, per_w)
    dst3 = dst_p.reshape(32, n_chunks, CHUNK)

    def sc_body(y_hbm, src_hbm, et_hbm, dst_hbm, out_hbm,
                src_v, et_v, key_v, dst_v, rows0_v, rows1_v, zrow_v, h_sh,
                sem0, sem1):
        cid = lax.axis_index("c")
        sid = lax.axis_index("s")
        wid = sid * 2 + cid

        for i in range(16):
            for j in range(OUT_FEAT // 16):
                zrow_v[i, pl.ds(j * 16, 16)] = jnp.zeros((16,), jnp.float32)

        def zero_body(k, _):
            pltpu.sync_copy(zrow_v, h_sh.at[pl.ds(sid * rows_per_tile + k * 16, 16)])
            return _
        lax.fori_loop(0, rows_per_tile // 16, zero_body, None)

        pltpu.sync_copy(dst_hbm.at[wid], dst_v)
        piece = per_w // N_PIECES

        def piece_body(p, _):
            pltpu.sync_copy(src_hbm.at[wid, pl.ds(p * piece, piece)], src_v)
            pltpu.sync_copy(et_hbm.at[wid, pl.ds(p * piece, piece)], et_v)

            def key_body(i, _):
                sl = pl.ds(i * 16, 16)
                key_v[pl.ds(p * piece + i * 16, 16)] = (
                    src_v[sl] * NUM_RELS + et_v[sl])
                return _
            lax.fori_loop(0, piece // 16, key_body, None)
            return _
        lax.fori_loop(0, N_PIECES, piece_body, None)
        plsc.subcore_barrier()

        bufs = (rows0_v, rows1_v)
        sems = (sem0, sem1)

        def start_gather(t, b):
            pltpu.async_copy(y_hbm.at[key_v.at[pl.ds(t * CHUNK, CHUNK)]],
                             bufs[b], sems[b])

        def wait_gather(b):
            pltpu.make_async_copy(y_hbm.at[key_v.at[pl.ds(0, CHUNK)]],
                                  bufs[b], sems[b]).wait()

        start_gather(0, 0)

        def pair_body(i, _):
            t0 = i * 2
            @pl.when(t0 + 1 < n_chunks)
            def _():
                start_gather(t0 + 1, 1)
            wait_gather(0)
            pltpu.sync_copy(bufs[0], h_sh.at[dst_v.at[t0]], add=True)

            @pl.when(t0 + 2 < n_chunks)
            def _():
                start_gather(t0 + 2, 0)
            @pl.when(t0 + 1 < n_chunks)
            def _():
                wait_gather(1)
                pltpu.sync_copy(bufs[1], h_sh.at[dst_v.at[t0 + 1]], add=True)
            return _
        lax.fori_loop(0, (n_chunks + 1) // 2, pair_body, None)
        plsc.subcore_barrier()

        pltpu.sync_copy(h_sh.at[pl.ds(sid * rows_per_tile, rows_per_tile)],
                        out_hbm.at[cid, pl.ds(sid * rows_per_tile, rows_per_tile)])

    partials = pl.kernel(
        sc_body,
        out_type=jax.ShapeDtypeStruct((2, H_ROWS, OUT_FEAT), jnp.float32),
        mesh=mesh,
        scratch_types=[
            pltpu.VMEM((per_w // N_PIECES,), jnp.int32),
            pltpu.VMEM((per_w // N_PIECES,), jnp.int32),
            pltpu.VMEM((per_w,), jnp.int32),
            pltpu.VMEM((n_chunks, CHUNK), jnp.int32),
            pltpu.VMEM((CHUNK, OUT_FEAT), jnp.float32),
            pltpu.VMEM((CHUNK, OUT_FEAT), jnp.float32),
            pltpu.VMEM((16, OUT_FEAT), jnp.float32),
            pltpu.VMEM_SHARED((H_ROWS, OUT_FEAT), jnp.float32),
            pltpu.SemaphoreType.DMA,
            pltpu.SemaphoreType.DMA,
        ],
    )(y_rows, src3, et3, dst3)

    bias2 = h_bias.reshape(1, OUT_FEAT)
    out = pl.pallas_call(
        _combine_body,
        grid=(n_nodes // CB_ROWS,),
        in_specs=[
            pl.BlockSpec((1, CB_ROWS, OUT_FEAT), lambda i: (0, i, 0)),
            pl.BlockSpec((1, CB_ROWS, OUT_FEAT), lambda i: (1, i, 0)),
            pl.BlockSpec((1, OUT_FEAT), lambda i: (0, 0)),
        ],
        out_specs=pl.BlockSpec((CB_ROWS, OUT_FEAT), lambda i: (i, 0)),
        out_shape=jax.ShapeDtypeStruct((n_nodes, OUT_FEAT), jnp.float32),
    )(partials, partials, bias2)
    return out
